# Initial kernel scaffold; baseline (speedup 1.0000x reference)
import jax
import jax.numpy as jnp
from jax.experimental import pallas as pl


def kernel(x, edge_index, T, T_max, W1, b1, W2, b2, W3, b3, P1, pb1, P2, pb2, P3, pb3):
    raise NotImplementedError("write your pallas kernel here")


# speedup vs baseline: 39.5127x; 39.5127x over previous
import jax
import jax.numpy as jnp
from jax import lax
from jax.experimental import pallas as pl
from jax.experimental.pallas import tpu as pltpu
from jax.experimental.pallas import tpu_sc as plsc

N = 100000
E = 3200000
CH = 128
NCH = E // CH
BLK = 16
NBLK = NCH // BLK
REM = NCH - NBLK * BLK
NPAD = 102400
PT = NPAD // 16
NC, NS = 2, 16
NW = NC * NS
RB = N // NS
RCH = 800
NRC = N // RCH

_mesh = None


def _get_mesh():
    global _mesh
    if _mesh is None:
        _mesh = plsc.VectorSubcoreMesh(core_axis_name="c", subcore_axis_name="s")
    return _mesh


def _wait(src, dst, sem):
    pltpu.make_async_copy(src, dst, sem).wait()


def _sc_deg_body(dst2d, out, zbuf, idxb, ones, accum, s0, s1, s2, s3):
    sems = [s0, s1, s2, s3]
    c = lax.axis_index("c")
    s = lax.axis_index("s")
    wid = s * NC + c

    @pl.loop(0, PT // 16)
    def _z(i):
        zbuf[pl.ds(i * 16, 16)] = jnp.zeros((16,), jnp.float32)

    for m in range(CH // 16):
        ones[pl.ds(m * 16, 16)] = jnp.ones((16,), jnp.float32)
    pltpu.sync_copy(zbuf, accum.at[pl.ds(s * PT, PT)])
    plsc.subcore_barrier()

    nblk_my = (NBLK - wid + NW - 1) // NW

    @pl.loop(0, nblk_my)
    def _blk(i):
        b0 = (wid + i * NW) * BLK
        pltpu.sync_copy(dst2d.at[pl.ds(b0, BLK)], idxb)
        for k in range(BLK):
            if k >= 4:
                _wait(ones, accum.at[idxb.at[k - 4]], sems[k % 4])
            pltpu.async_copy(ones, accum.at[idxb.at[k]], sems[k % 4], add=True)
        for k in range(BLK - 4, BLK):
            _wait(ones, accum.at[idxb.at[k]], sems[k % 4])

    @pl.when(wid < REM)
    def _rem():
        pltpu.sync_copy(dst2d.at[pl.ds(NBLK * BLK + wid, 1)], idxb.at[pl.ds(0, 1)])
        pltpu.sync_copy(ones, accum.at[idxb.at[0]], add=True)

    plsc.subcore_barrier()
    pltpu.sync_copy(accum.at[pl.ds(s * PT, PT)], out.at[c].at[pl.ds(s * PT, PT)])


def _sc_deg(dst2d):
    return pl.kernel(
        _sc_deg_body,
        out_type=jax.ShapeDtypeStruct((NC, NPAD), jnp.float32),
        mesh=_get_mesh(),
        compiler_params=pltpu.CompilerParams(use_tc_tiling_on_sc=False),
        scratch_types=[
            pltpu.VMEM((PT,), jnp.float32),
            pltpu.VMEM((BLK, CH), jnp.int32),
            pltpu.VMEM((CH,), jnp.float32),
            pltpu.VMEM_SHARED((NPAD,), jnp.float32),
        ] + [pltpu.SemaphoreType.DMA] * 4,
    )(dst2d)


def _make_sc_l1():
    def body(table0, src2d, dst2d, dinv1d, zeros16, aggp, sp,
             idxS, idxD, r0, r1, r2, r3, v0, v1, v2, v3, zbuf,
             dinv_sh, acc16, sacc,
             g0, g1, g2, g3, c0s, c1s, c2s, c3s, e0, e1, e2, e3,
             f0, f1, f2, f3):
        rows = [r0, r1, r2, r3]
        valr = [v0, v1, v2, v3]
        gsem = [g0, g1, g2, g3]
        csem = [c0s, c1s, c2s, c3s]
        esem = [e0, e1, e2, e3]
        vsem = [f0, f1, f2, f3]
        c = lax.axis_index("c")
        s = lax.axis_index("s")
        wid = s * NC + c

        @pl.when(s == 0)
        def _stage():
            pltpu.sync_copy(dinv1d, dinv_sh)

        @pl.loop(0, PT // 16)
        def _z(i):
            zbuf[pl.ds(i * 16, 16)] = jnp.zeros((16,), jnp.float32)

        pltpu.sync_copy(zbuf, sacc.at[pl.ds(s * PT, PT)])

        @pl.loop(0, (NRC - s + NS - 1) // NS)
        def _init(i):
            r0 = (s + i * NS) * RCH
            pltpu.sync_copy(zeros16.at[pl.ds(r0, RCH)], acc16.at[pl.ds(r0, RCH)])

        plsc.subcore_barrier()

        def do_chunk_pipe(idxS_ref, idxD_ref):
            for k in range(BLK + 3):
                if k < BLK:
                    if k >= 4:
                        _wait(rows[k % 4], acc16.at[idxD_ref.at[k - 4]], csem[k % 4])
                        _wait(valr[k % 4], sacc.at[idxS_ref.at[k - 4]], esem[k % 4])
                    pltpu.async_copy(dinv_sh.at[idxD_ref.at[k]], valr[k % 4],
                                     vsem[k % 4])
                    pltpu.async_copy(table0.at[idxS_ref.at[k]], rows[k % 4],
                                     gsem[k % 4])
                if k >= 3:
                    kk = k - 3
                    _wait(table0.at[idxS_ref.at[kk]], rows[kk % 4], gsem[kk % 4])
                    pltpu.async_copy(rows[kk % 4], acc16.at[idxD_ref.at[kk]],
                                     csem[kk % 4], add=True)
                    _wait(dinv_sh.at[idxD_ref.at[kk]], valr[kk % 4], vsem[kk % 4])
                    pltpu.async_copy(valr[kk % 4], sacc.at[idxS_ref.at[kk]],
                                     esem[kk % 4], add=True)
            for k in range(BLK - 4, BLK):
                _wait(rows[k % 4], acc16.at[idxD_ref.at[k]], csem[k % 4])
                _wait(valr[k % 4], sacc.at[idxS_ref.at[k]], esem[k % 4])

        nblk_my = (NBLK - wid + NW - 1) // NW

        @pl.loop(0, nblk_my)
        def _blk(i):
            b0 = (wid + i * NW) * BLK
            pltpu.sync_copy(src2d.at[pl.ds(b0, BLK)], idxS)
            pltpu.sync_copy(dst2d.at[pl.ds(b0, BLK)], idxD)
            do_chunk_pipe(idxS, idxD)

        @pl.when(wid < REM)
        def _rem():
            cix = NBLK * BLK + wid
            pltpu.sync_copy(src2d.at[pl.ds(cix, 1)], idxS.at[pl.ds(0, 1)])
            pltpu.sync_copy(dst2d.at[pl.ds(cix, 1)], idxD.at[pl.ds(0, 1)])
            pltpu.async_copy(dinv_sh.at[idxD.at[0]], valr[0], vsem[0]).wait()
            pltpu.sync_copy(valr[0], sacc.at[idxS.at[0]], add=True)
            pltpu.async_copy(table0.at[idxS.at[0]], rows[0], gsem[0]).wait()
            pltpu.sync_copy(rows[0], acc16.at[idxD.at[0]], add=True)

        plsc.subcore_barrier()

        @pl.loop(0, (NRC - s + NS - 1) // NS)
        def _dump(i):
            r0 = (s + i * NS) * RCH
            pltpu.sync_copy(acc16.at[pl.ds(r0, RCH)],
                            aggp.at[c].at[pl.ds(r0, RCH)])

        pltpu.sync_copy(sacc.at[pl.ds(s * PT, PT)], sp.at[c].at[pl.ds(s * PT, PT)])

    return pl.kernel(
        body,
        out_type=(jax.ShapeDtypeStruct((NC, N, 8), jnp.float32),
                  jax.ShapeDtypeStruct((NC, NPAD), jnp.float32)),
        mesh=_get_mesh(),
        compiler_params=pltpu.CompilerParams(use_tc_tiling_on_sc=False),
        scratch_types=[
            pltpu.VMEM((BLK, CH), jnp.int32),
            pltpu.VMEM((BLK, CH), jnp.int32),
        ] + [pltpu.VMEM((CH, 8), jnp.float32)] * 4
          + [pltpu.VMEM((CH,), jnp.float32)] * 4
          + [
            pltpu.VMEM((PT,), jnp.float32),
            pltpu.VMEM_SHARED((N,), jnp.float32),
            pltpu.VMEM_SHARED((N, 8), jnp.float32),
            pltpu.VMEM_SHARED((NPAD,), jnp.float32),
        ] + [pltpu.SemaphoreType.DMA] * 16,
    )


def _make_sc_l2():
    def body(t0, t1, t2, t3, src2d, dst2d, out,
             idxS, idxD, r0, r1, r2, r3, acc16,
             g0, g1, g2, g3, c0s, c1s, c2s, c3s):
        rows = [r0, r1, r2, r3]
        gsem = [g0, g1, g2, g3]
        csem = [c0s, c1s, c2s, c3s]
        c = lax.axis_index("c")
        s = lax.axis_index("s")
        tabs = [t0, t1, t2, t3]

        for g in range(4):
            tref = tabs[g]

            @pl.when(c == g // 2)
            def _grp(tref=tref, g=g):
                @pl.loop(0, (NRC - s + NS - 1) // NS)
                def _init(i):
                    r0 = (s + i * NS) * RCH
                    pltpu.sync_copy(tref.at[pl.ds(r0, RCH)],
                                    acc16.at[pl.ds(r0, RCH)])

                plsc.subcore_barrier()

                nblk_my = (NBLK - s + NS - 1) // NS

                @pl.loop(0, nblk_my)
                def _blk(i):
                    b0 = (s + i * NS) * BLK
                    pltpu.sync_copy(src2d.at[pl.ds(b0, BLK)], idxS)
                    pltpu.sync_copy(dst2d.at[pl.ds(b0, BLK)], idxD)
                    for k in range(BLK + 3):
                        if k < BLK:
                            if k >= 4:
                                _wait(rows[k % 4], acc16.at[idxD.at[k - 4]],
                                      csem[k % 4])
                            pltpu.async_copy(tref.at[idxS.at[k]], rows[k % 4],
                                             gsem[k % 4])
                        if k >= 3:
                            kk = k - 3
                            _wait(tref.at[idxS.at[kk]], rows[kk % 4], gsem[kk % 4])
                            pltpu.async_copy(rows[kk % 4], acc16.at[idxD.at[kk]],
                                             csem[kk % 4], add=True)
                    for k in range(BLK - 4, BLK):
                        _wait(rows[k % 4], acc16.at[idxD.at[k]], csem[k % 4])

                @pl.when(s < REM)
                def _rem():
                    cix = NBLK * BLK + s
                    pltpu.sync_copy(src2d.at[pl.ds(cix, 1)], idxS.at[pl.ds(0, 1)])
                    pltpu.sync_copy(dst2d.at[pl.ds(cix, 1)], idxD.at[pl.ds(0, 1)])
                    pltpu.async_copy(tref.at[idxS.at[0]], rows[0], gsem[0]).wait()
                    pltpu.sync_copy(rows[0], acc16.at[idxD.at[0]], add=True)

                plsc.subcore_barrier()

                @pl.loop(0, (NRC - s + NS - 1) // NS)
                def _dump(i):
                    r0 = (s + i * NS) * RCH
                    pltpu.sync_copy(acc16.at[pl.ds(r0, RCH)],
                                    out.at[g].at[pl.ds(r0, RCH)])

    return pl.kernel(
        body,
        out_type=jax.ShapeDtypeStruct((4, N, 16), jnp.float32),
        mesh=_get_mesh(),
        compiler_params=pltpu.CompilerParams(use_tc_tiling_on_sc=False),
        scratch_types=[
            pltpu.VMEM((BLK, CH), jnp.int32),
            pltpu.VMEM((BLK, CH), jnp.int32),
        ] + [pltpu.VMEM((CH, 16), jnp.float32)] * 4
          + [pltpu.VMEM_SHARED((N, 16), jnp.float32)]
          + [pltpu.SemaphoreType.DMA] * 8,
    )


B = 2000
GRID = N // B


def _tc_pre_body(x_ref, degp_ref, tab_ref, stats_ref):
    i = pl.program_id(0)
    xb = x_ref[...]
    deg = degp_ref[:, 0] + degp_ref[:, 1] + 1.0
    dinv = lax.rsqrt(deg)[:, None]
    tab = jnp.concatenate(
        [xb * dinv, dinv, jnp.zeros((B, 2), jnp.float32)], axis=1)
    tab_ref[...] = tab
    m = (xb[:, 2] == 1.0).astype(jnp.float32)
    vals = jnp.stack([
        jnp.sum(xb[:, 2]), jnp.sum(xb[:, 3]), jnp.sum(xb[:, 4]),
        jnp.sum(xb[:, 0] * m), jnp.sum(xb[:, 1] * m), jnp.sum(m),
        jnp.float32(0.0), jnp.float32(0.0)])[None, :]

    @pl.when(i == 0)
    def _():
        stats_ref[...] = vals

    @pl.when(i != 0)
    def _():
        stats_ref[...] = stats_ref[...] + vals


def _tc_pre(x, degp2):
    return pl.pallas_call(
        _tc_pre_body,
        grid=(GRID,),
        in_specs=[
            pl.BlockSpec((B, 5), lambda i: (i, 0)),
            pl.BlockSpec((B, NC), lambda i: (i, 0)),
        ],
        out_specs=[
            pl.BlockSpec((B, 8), lambda i: (i, 0)),
            pl.BlockSpec((1, 8), lambda i: (0, 0)),
        ],
        out_shape=[
            jax.ShapeDtypeStruct((N, 8), jnp.float32),
            jax.ShapeDtypeStruct((1, 8), jnp.float32),
        ],
    )(x, degp2)


def _tc_mid_body(aggp_ref, tab_ref, w1_ref, b1_ref, o0, o1, o2, o3):
    tab = tab_ref[...]
    agg = aggp_ref[0] + aggp_ref[1] + tab
    dinv = tab[:, 5:6]
    z = agg * dinv
    h1 = jnp.maximum(jnp.dot(z, w1_ref[...],
                             preferred_element_type=jnp.float32)
                     + b1_ref[...], 0.0)
    y1 = h1 * dinv
    o0[...] = y1[:, 0:16]
    o1[...] = y1[:, 16:32]
    o2[...] = y1[:, 32:48]
    o3[...] = y1[:, 48:64]


def _tc_mid(aggp, tab, w1p, b1_2d):
    return pl.pallas_call(
        _tc_mid_body,
        grid=(GRID,),
        in_specs=[
            pl.BlockSpec((NC, B, 8), lambda i: (0, i, 0)),
            pl.BlockSpec((B, 8), lambda i: (i, 0)),
            pl.BlockSpec((8, 64), lambda i: (0, 0)),
            pl.BlockSpec((1, 64), lambda i: (0, 0)),
        ],
        out_specs=[pl.BlockSpec((B, 16), lambda i: (i, 0))] * 4,
        out_shape=[jax.ShapeDtypeStruct((N, 16), jnp.float32)] * 4,
    )(aggp, tab, w1p, b1_2d)


def _tc_post_body(g0, g1, g2, g3, tab_ref, sp_ref, w2_ref, b2_ref, ev_ref):
    i = pl.program_id(0)
    agg = jnp.concatenate([g0[...], g1[...], g2[...], g3[...]], axis=1)
    tab = tab_ref[...]
    dinv = tab[:, 5:6]
    h2 = jnp.maximum(jnp.dot(agg * dinv, w2_ref[...],
                             preferred_element_type=jnp.float32)
                     + b2_ref[...], 0.0)
    sv = sp_ref[:, 0] + sp_ref[:, 1]
    cvec = (dinv[:, 0] * (sv + dinv[:, 0]))[None, :]
    contrib = jnp.dot(cvec, h2, preferred_element_type=jnp.float32)

    @pl.when(i == 0)
    def _():
        ev_ref[...] = contrib

    @pl.when(i != 0)
    def _():
        ev_ref[...] = ev_ref[...] + contrib


def _tc_post(a0, a1, a2, a3, tab, sp2, w2, b2_2d):
    return pl.pallas_call(
        _tc_post_body,
        grid=(GRID,),
        in_specs=[pl.BlockSpec((B, 16), lambda i: (i, 0))] * 4 + [
            pl.BlockSpec((B, 8), lambda i: (i, 0)),
            pl.BlockSpec((B, NC), lambda i: (i, 0)),
            pl.BlockSpec((64, 64), lambda i: (0, 0)),
            pl.BlockSpec((1, 64), lambda i: (0, 0)),
        ],
        out_specs=pl.BlockSpec((1, 64), lambda i: (0, 0)),
        out_shape=jax.ShapeDtypeStruct((1, 64), jnp.float32),
    )(a0, a1, a2, a3, tab, sp2, w2, b2_2d)


def _tc_head_body(ev, stats, tn, w3, b3, p1, pb1, p2, pb2, p3, pb3,
                  out_ref, emb_ref):
    emb0 = jnp.dot(ev[...] * (1.0 / N), w3[...],
                   preferred_element_type=jnp.float32) + b3[...]
    st = stats[...]
    n_comp, n_and, n_or = st[0, 0], st[0, 1], st[0, 2]
    cnt = st[0, 5]
    avg_l = jnp.where(cnt > 0, st[0, 3] / jnp.maximum(cnt, 1.0), 0.0)
    avg_m = jnp.where(cnt > 0, st[0, 4] / jnp.maximum(cnt, 1.0), 0.0)
    tnv = tn[0, 0]
    gf = jnp.stack([n_comp, n_and, n_or, n_and + n_or, avg_l, avg_m,
                    tnv * 50.0, (1.0 / (1.0 + tnv)) * 50.0])[None, :]
    emb = jnp.concatenate([emb0, gf], axis=1)
    emb_ref[...] = emb
    h = jnp.maximum(jnp.dot(emb, p1[...], preferred_element_type=jnp.float32)
                    + pb1[...], 0.0)
    h = jnp.maximum(jnp.dot(h, p2[...], preferred_element_type=jnp.float32)
                    + pb2[...], 0.0)
    raw = jnp.dot(h, p3[...], preferred_element_type=jnp.float32) + pb3[...]
    z = raw + 2.0
    val = jnp.maximum(z, 0.0) + jnp.log1p(jnp.exp(-jnp.abs(z)))
    amin = 1.0 + val[:, 0:1]
    amax = amin + val[:, 1:2] + 0.5
    bmin = 1.0 + val[:, 2:3]
    bmax = bmin + val[:, 3:4] + 0.5
    out_ref[...] = jnp.concatenate([amin, amax, bmin, bmax], axis=1)


def _tc_head(ev, stats, tn, w3, b3_2d, p1, pb1_2d, p2, pb2_2d, p3, pb3_2d):
    return pl.pallas_call(
        _tc_head_body,
        out_shape=[
            jax.ShapeDtypeStruct((1, 4), jnp.float32),
            jax.ShapeDtypeStruct((1, 24), jnp.float32),
        ],
    )(ev, stats, tn, w3, b3_2d, p1, pb1_2d, p2, pb2_2d, p3, pb3_2d)


def kernel(x, edge_index, T, T_max, W1, b1, W2, b2, W3, b3,
           P1, pb1, P2, pb2, P3, pb3):
    src2d = edge_index[0].reshape(NCH, CH)
    dst2d = edge_index[1].reshape(NCH, CH)

    degp = _sc_deg(dst2d)
    degp2 = degp[:, :N].T

    tab, stats = _tc_pre(x, degp2)
    dinv1d = tab[:, 5]
    dinv1d = jnp.asarray(dinv1d, jnp.float32).reshape(N)

    zeros16 = jnp.zeros((N, 8), jnp.float32)
    aggp, sp = _make_sc_l1()(tab, src2d, dst2d, dinv1d, zeros16)

    w1p = jnp.zeros((8, 64), jnp.float32).at[:5].set(W1)
    t0, t1, t2, t3 = _tc_mid(aggp, tab, w1p, b1.reshape(1, 64))

    agg2 = _make_sc_l2()(t0, t1, t2, t3, src2d, dst2d)

    ev = _tc_post(agg2[0], agg2[1], agg2[2], agg2[3], tab, sp[:, :N].T,
                  W2, b2.reshape(1, 64))

    tn = (T / T_max) * jnp.ones((1, 1), jnp.float32)
    out4, emb = _tc_head(ev, stats, tn, W3, b3.reshape(1, 16),
                         P1, pb1.reshape(1, 64), P2, pb2.reshape(1, 32),
                         P3, pb3.reshape(1, 4))
    return (out4, emb)


# speedup vs baseline: 43.4728x; 1.1002x over previous
import jax
import jax.numpy as jnp
from jax import lax
from jax.experimental import pallas as pl
from jax.experimental.pallas import tpu as pltpu
from jax.experimental.pallas import tpu_sc as plsc

N = 100000
E = 3200000
CH = 128
SCW = 2
NSCH = E // (SCW * CH)
BLK = 8
NBLK = NSCH // BLK
REM = NSCH - NBLK * BLK
NPAD = 102400
PT = NPAD // 16
NC, NS = 2, 16
NW = NC * NS
RCH = 800
NRC = N // RCH

_mesh = None


def _get_mesh():
    global _mesh
    if _mesh is None:
        _mesh = plsc.VectorSubcoreMesh(core_axis_name="c", subcore_axis_name="s")
    return _mesh


def _wait(src, dst, sem):
    pltpu.make_async_copy(src, dst, sem).wait()


def _sc_deg_body(dst3d, out, zbuf, idxb, ones, accum, s0, s1, s2, s3):
    sems = [s0, s1, s2, s3]
    c = lax.axis_index("c")
    s = lax.axis_index("s")
    wid = s * NC + c

    @pl.loop(0, PT // 16)
    def _z(i):
        zbuf[pl.ds(i * 16, 16)] = jnp.zeros((16,), jnp.float32)

    for m in range(SCW * CH // 16):
        ones[pl.ds(m * 16, 16)] = jnp.ones((16,), jnp.float32)

    ones2 = ones
    pltpu.sync_copy(zbuf, accum.at[pl.ds(s * PT, PT)])
    plsc.subcore_barrier()

    nblk_my = (NBLK - wid + NW - 1) // NW

    @pl.loop(0, nblk_my)
    def _blk(i):
        b0 = (wid + i * NW) * BLK
        pltpu.sync_copy(dst3d.at[pl.ds(b0, BLK)], idxb)
        for k in range(BLK):
            if k >= 4:
                _wait(ones2, accum.at[idxb.at[k - 4]], sems[k % 4])
            pltpu.async_copy(ones2, accum.at[idxb.at[k]], sems[k % 4], add=True)
        for k in range(BLK - 4, BLK):
            _wait(ones2, accum.at[idxb.at[k]], sems[k % 4])

    @pl.when(wid < REM)
    def _rem():
        pltpu.sync_copy(dst3d.at[NBLK * BLK + wid], idxb.at[0])
        pltpu.sync_copy(ones2, accum.at[idxb.at[0]], add=True)

    plsc.subcore_barrier()
    pltpu.sync_copy(accum.at[pl.ds(s * PT, PT)], out.at[c].at[pl.ds(s * PT, PT)])


def _sc_deg(dst3d):
    return pl.kernel(
        _sc_deg_body,
        out_type=jax.ShapeDtypeStruct((NC, NPAD), jnp.float32),
        mesh=_get_mesh(),
        compiler_params=pltpu.CompilerParams(use_tc_tiling_on_sc=False),
        scratch_types=[
            pltpu.VMEM((PT,), jnp.float32),
            pltpu.VMEM((BLK, SCW * CH), jnp.int32),
            pltpu.VMEM((SCW * CH,), jnp.float32),
            pltpu.VMEM_SHARED((NPAD,), jnp.float32),
        ] + [pltpu.SemaphoreType.DMA] * 4,
    )(dst3d)


def _make_sc_l1():
    def body(table0, src3d, dst3d, dinv1d, zeros8, aggp, sp,
             bufS, bufD, r0, r1, r2, r3, v0, v1, v2, v3, zbuf,
             dinv_sh, acc, sacc,
             g0, g1, g2, g3, c0s, c1s, c2s, c3s, e0, e1, e2, e3,
             f0, f1, f2, f3):
        rows = [r0, r1, r2, r3]
        valr = [v0, v1, v2, v3]
        gsem = [g0, g1, g2, g3]
        csem = [c0s, c1s, c2s, c3s]
        esem = [e0, e1, e2, e3]
        vsem = [f0, f1, f2, f3]
        c = lax.axis_index("c")
        s = lax.axis_index("s")
        wid = s * NC + c

        @pl.when(s == 0)
        def _stage():
            pltpu.sync_copy(dinv1d, dinv_sh)

        @pl.loop(0, PT // 16)
        def _z(i):
            zbuf[pl.ds(i * 16, 16)] = jnp.zeros((16,), jnp.float32)

        pltpu.sync_copy(zbuf, sacc.at[pl.ds(s * PT, PT)])

        @pl.loop(0, (NRC - s + NS - 1) // NS)
        def _init(i):
            r0_ = (s + i * NS) * RCH
            pltpu.sync_copy(zeros8.at[pl.ds(r0_, RCH)], acc.at[pl.ds(r0_, RCH)])

        plsc.subcore_barrier()

        def pipe(iS, iD):
            for k in range(BLK + 3):
                if k < BLK:
                    if k >= 4:
                        _wait(rows[k % 4], acc.at[iD.at[k - 4]], csem[k % 4])
                        _wait(valr[k % 4], sacc.at[iS.at[k - 4]], esem[k % 4])
                    pltpu.async_copy(dinv_sh.at[iD.at[k]], valr[k % 4],
                                     vsem[k % 4])
                    pltpu.async_copy(table0.at[iS.at[k]], rows[k % 4],
                                     gsem[k % 4])
                if k >= 3:
                    kk = k - 3
                    _wait(table0.at[iS.at[kk]], rows[kk % 4], gsem[kk % 4])
                    pltpu.async_copy(rows[kk % 4], acc.at[iD.at[kk]],
                                     csem[kk % 4], add=True)
                    _wait(dinv_sh.at[iD.at[kk]], valr[kk % 4], vsem[kk % 4])
                    pltpu.async_copy(valr[kk % 4], sacc.at[iS.at[kk]],
                                     esem[kk % 4], add=True)
            for k in range(BLK - 4, BLK):
                _wait(rows[k % 4], acc.at[iD.at[k]], csem[k % 4])
                _wait(valr[k % 4], sacc.at[iS.at[k]], esem[k % 4])

        nblk_my = (NBLK - wid + NW - 1) // NW

        @pl.loop(0, nblk_my)
        def _blk(i):
            b0 = (wid + i * NW) * BLK
            pltpu.sync_copy(src3d.at[pl.ds(b0, BLK)], bufS)
            pltpu.sync_copy(dst3d.at[pl.ds(b0, BLK)], bufD)
            pipe(bufS, bufD)

        @pl.when(wid < REM)
        def _rem():
            cix = NBLK * BLK + wid
            pltpu.sync_copy(src3d.at[cix], bufS.at[0])
            pltpu.sync_copy(dst3d.at[cix], bufD.at[0])
            pltpu.async_copy(dinv_sh.at[bufD.at[0]], valr[0], vsem[0]).wait()
            pltpu.sync_copy(valr[0], sacc.at[bufS.at[0]], add=True)
            pltpu.async_copy(table0.at[bufS.at[0]], rows[0], gsem[0]).wait()
            pltpu.sync_copy(rows[0], acc.at[bufD.at[0]], add=True)

        plsc.subcore_barrier()

        @pl.loop(0, (NRC - s + NS - 1) // NS)
        def _dump(i):
            r0_ = (s + i * NS) * RCH
            pltpu.sync_copy(acc.at[pl.ds(r0_, RCH)],
                            aggp.at[c].at[pl.ds(r0_, RCH)])

        pltpu.sync_copy(sacc.at[pl.ds(s * PT, PT)], sp.at[c].at[pl.ds(s * PT, PT)])

    return pl.kernel(
        body,
        out_type=(jax.ShapeDtypeStruct((NC, N, 8), jnp.float32),
                  jax.ShapeDtypeStruct((NC, NPAD), jnp.float32)),
        mesh=_get_mesh(),
        compiler_params=pltpu.CompilerParams(use_tc_tiling_on_sc=False),
        scratch_types=[
            pltpu.VMEM((BLK, SCW * CH), jnp.int32),
            pltpu.VMEM((BLK, SCW * CH), jnp.int32),
        ] + [pltpu.VMEM((SCW * CH, 8), jnp.float32)] * 4
          + [pltpu.VMEM((SCW * CH,), jnp.float32)] * 4
          + [
            pltpu.VMEM((PT,), jnp.float32),
            pltpu.VMEM_SHARED((N,), jnp.float32),
            pltpu.VMEM_SHARED((N, 8), jnp.float32),
            pltpu.VMEM_SHARED((NPAD,), jnp.float32),
        ] + [pltpu.SemaphoreType.DMA] * 16,
    )


def _make_sc_l2():
    def body(t0, t1, t2, t3, src3d, dst3d, out,
             bufS, bufD, r0, r1, r2, r3, acc,
             g0, g1, g2, g3, c0s, c1s, c2s, c3s):
        rows = [r0, r1, r2, r3]
        gsem = [g0, g1, g2, g3]
        csem = [c0s, c1s, c2s, c3s]
        c = lax.axis_index("c")
        s = lax.axis_index("s")
        tabs = [t0, t1, t2, t3]

        for g in range(4):
            tref = tabs[g]

            @pl.when(c == g // 2)
            def _grp(tref=tref, g=g):
                @pl.loop(0, (NRC - s + NS - 1) // NS)
                def _init(i):
                    r0_ = (s + i * NS) * RCH
                    pltpu.sync_copy(tref.at[pl.ds(r0_, RCH)],
                                    acc.at[pl.ds(r0_, RCH)])

                plsc.subcore_barrier()

                nblk_my = (NBLK - s + NS - 1) // NS

                @pl.loop(0, nblk_my)
                def _blk(i):
                    b0 = (s + i * NS) * BLK
                    pltpu.sync_copy(src3d.at[pl.ds(b0, BLK)], bufS)
                    pltpu.sync_copy(dst3d.at[pl.ds(b0, BLK)], bufD)
                    for k in range(BLK + 3):
                        if k < BLK:
                            if k >= 4:
                                _wait(rows[k % 4], acc.at[bufD.at[k - 4]],
                                      csem[k % 4])
                            pltpu.async_copy(tref.at[bufS.at[k]], rows[k % 4],
                                             gsem[k % 4])
                        if k >= 3:
                            kk = k - 3
                            _wait(tref.at[bufS.at[kk]], rows[kk % 4], gsem[kk % 4])
                            pltpu.async_copy(rows[kk % 4], acc.at[bufD.at[kk]],
                                             csem[kk % 4], add=True)
                    for k in range(BLK - 4, BLK):
                        _wait(rows[k % 4], acc.at[bufD.at[k]], csem[k % 4])

                @pl.when(s < REM)
                def _rem():
                    cix = NBLK * BLK + s
                    pltpu.sync_copy(src3d.at[cix], bufS.at[0])
                    pltpu.sync_copy(dst3d.at[cix], bufD.at[0])
                    pltpu.async_copy(tref.at[bufS.at[0]], rows[0], gsem[0]).wait()
                    pltpu.sync_copy(rows[0], acc.at[bufD.at[0]], add=True)

                plsc.subcore_barrier()

                @pl.loop(0, (NRC - s + NS - 1) // NS)
                def _dump(i):
                    r0_ = (s + i * NS) * RCH
                    pltpu.sync_copy(acc.at[pl.ds(r0_, RCH)],
                                    out.at[g].at[pl.ds(r0_, RCH)])

    return pl.kernel(
        body,
        out_type=jax.ShapeDtypeStruct((4, N, 16), jnp.float32),
        mesh=_get_mesh(),
        compiler_params=pltpu.CompilerParams(use_tc_tiling_on_sc=False,
                                             internal_scratch_in_bytes=1024 * 1024),
        scratch_types=[
            pltpu.VMEM((BLK, SCW * CH), jnp.int32),
            pltpu.VMEM((BLK, SCW * CH), jnp.int32),
        ] + [pltpu.VMEM((SCW * CH, 16), jnp.float32)] * 4
          + [pltpu.VMEM_SHARED((N, 16), jnp.float32)]
          + [pltpu.SemaphoreType.DMA] * 8,
    )


B = 2000
GRID = N // B


def _tc_pre_body(x_ref, degp_ref, tab_ref, stats_ref):
    i = pl.program_id(0)
    xb = x_ref[...]
    deg = degp_ref[:, 0] + degp_ref[:, 1] + 1.0
    dinv = lax.rsqrt(deg)[:, None]
    tab = jnp.concatenate(
        [xb * dinv, dinv, jnp.zeros((B, 2), jnp.float32)], axis=1)
    tab_ref[...] = tab
    m = (xb[:, 2] == 1.0).astype(jnp.float32)
    vals = jnp.stack([
        jnp.sum(xb[:, 2]), jnp.sum(xb[:, 3]), jnp.sum(xb[:, 4]),
        jnp.sum(xb[:, 0] * m), jnp.sum(xb[:, 1] * m), jnp.sum(m),
        jnp.float32(0.0), jnp.float32(0.0)])[None, :]

    @pl.when(i == 0)
    def _():
        stats_ref[...] = vals

    @pl.when(i != 0)
    def _():
        stats_ref[...] = stats_ref[...] + vals


def _tc_pre(x, degp2):
    return pl.pallas_call(
        _tc_pre_body,
        grid=(GRID,),
        in_specs=[
            pl.BlockSpec((B, 5), lambda i: (i, 0)),
            pl.BlockSpec((B, NC), lambda i: (i, 0)),
        ],
        out_specs=[
            pl.BlockSpec((B, 8), lambda i: (i, 0)),
            pl.BlockSpec((1, 8), lambda i: (0, 0)),
        ],
        out_shape=[
            jax.ShapeDtypeStruct((N, 8), jnp.float32),
            jax.ShapeDtypeStruct((1, 8), jnp.float32),
        ],
    )(x, degp2)


def _tc_mid_body(aggp_ref, tab_ref, w1_ref, b1_ref, o0, o1, o2, o3):
    tab = tab_ref[...]
    agg = aggp_ref[0] + aggp_ref[1] + tab
    dinv = tab[:, 5:6]
    z = agg * dinv
    h1 = jnp.maximum(jnp.dot(z, w1_ref[...],
                             preferred_element_type=jnp.float32)
                     + b1_ref[...], 0.0)
    y1 = h1 * dinv
    o0[...] = y1[:, 0:16]
    o1[...] = y1[:, 16:32]
    o2[...] = y1[:, 32:48]
    o3[...] = y1[:, 48:64]


def _tc_mid(aggp, tab, w1p, b1_2d):
    return pl.pallas_call(
        _tc_mid_body,
        grid=(GRID,),
        in_specs=[
            pl.BlockSpec((NC, B, 8), lambda i: (0, i, 0)),
            pl.BlockSpec((B, 8), lambda i: (i, 0)),
            pl.BlockSpec((8, 64), lambda i: (0, 0)),
            pl.BlockSpec((1, 64), lambda i: (0, 0)),
        ],
        out_specs=[pl.BlockSpec((B, 16), lambda i: (i, 0))] * 4,
        out_shape=[jax.ShapeDtypeStruct((N, 16), jnp.float32)] * 4,
    )(aggp, tab, w1p, b1_2d)


def _tc_post_body(g0, g1, g2, g3, tab_ref, sp_ref, w2_ref, b2_ref, ev_ref):
    i = pl.program_id(0)
    agg = jnp.concatenate([g0[...], g1[...], g2[...], g3[...]], axis=1)
    tab = tab_ref[...]
    dinv = tab[:, 5:6]
    h2 = jnp.maximum(jnp.dot(agg * dinv, w2_ref[...],
                             preferred_element_type=jnp.float32)
                     + b2_ref[...], 0.0)
    sv = sp_ref[:, 0] + sp_ref[:, 1]
    cvec = (dinv[:, 0] * (sv + dinv[:, 0]))[None, :]
    contrib = jnp.dot(cvec, h2, preferred_element_type=jnp.float32)

    @pl.when(i == 0)
    def _():
        ev_ref[...] = contrib

    @pl.when(i != 0)
    def _():
        ev_ref[...] = ev_ref[...] + contrib


def _tc_post(a0, a1, a2, a3, tab, sp2, w2, b2_2d):
    return pl.pallas_call(
        _tc_post_body,
        grid=(GRID,),
        in_specs=[pl.BlockSpec((B, 16), lambda i: (i, 0))] * 4 + [
            pl.BlockSpec((B, 8), lambda i: (i, 0)),
            pl.BlockSpec((B, NC), lambda i: (i, 0)),
            pl.BlockSpec((64, 64), lambda i: (0, 0)),
            pl.BlockSpec((1, 64), lambda i: (0, 0)),
        ],
        out_specs=pl.BlockSpec((1, 64), lambda i: (0, 0)),
        out_shape=jax.ShapeDtypeStruct((1, 64), jnp.float32),
    )(a0, a1, a2, a3, tab, sp2, w2, b2_2d)


def _tc_head_body(ev, stats, tn, w3, b3, p1, pb1, p2, pb2, p3, pb3,
                  out_ref, emb_ref):
    emb0 = jnp.dot(ev[...] * (1.0 / N), w3[...],
                   preferred_element_type=jnp.float32) + b3[...]
    st = stats[...]
    n_comp, n_and, n_or = st[0, 0], st[0, 1], st[0, 2]
    cnt = st[0, 5]
    avg_l = jnp.where(cnt > 0, st[0, 3] / jnp.maximum(cnt, 1.0), 0.0)
    avg_m = jnp.where(cnt > 0, st[0, 4] / jnp.maximum(cnt, 1.0), 0.0)
    tnv = tn[0, 0]
    gf = jnp.stack([n_comp, n_and, n_or, n_and + n_or, avg_l, avg_m,
                    tnv * 50.0, (1.0 / (1.0 + tnv)) * 50.0])[None, :]
    emb = jnp.concatenate([emb0, gf], axis=1)
    emb_ref[...] = emb
    h = jnp.maximum(jnp.dot(emb, p1[...], preferred_element_type=jnp.float32)
                    + pb1[...], 0.0)
    h = jnp.maximum(jnp.dot(h, p2[...], preferred_element_type=jnp.float32)
                    + pb2[...], 0.0)
    raw = jnp.dot(h, p3[...], preferred_element_type=jnp.float32) + pb3[...]
    z = raw + 2.0
    val = jnp.maximum(z, 0.0) + jnp.log1p(jnp.exp(-jnp.abs(z)))
    amin = 1.0 + val[:, 0:1]
    amax = amin + val[:, 1:2] + 0.5
    bmin = 1.0 + val[:, 2:3]
    bmax = bmin + val[:, 3:4] + 0.5
    out_ref[...] = jnp.concatenate([amin, amax, bmin, bmax], axis=1)


def _tc_head(ev, stats, tn, w3, b3_2d, p1, pb1_2d, p2, pb2_2d, p3, pb3_2d):
    return pl.pallas_call(
        _tc_head_body,
        out_shape=[
            jax.ShapeDtypeStruct((1, 4), jnp.float32),
            jax.ShapeDtypeStruct((1, 24), jnp.float32),
        ],
    )(ev, stats, tn, w3, b3_2d, p1, pb1_2d, p2, pb2_2d, p3, pb3_2d)


def kernel(x, edge_index, T, T_max, W1, b1, W2, b2, W3, b3,
           P1, pb1, P2, pb2, P3, pb3):
    src3d = edge_index[0].reshape(NSCH, SCW * CH)
    dst3d = edge_index[1].reshape(NSCH, SCW * CH)

    degp = _sc_deg(dst3d)
    degp2 = degp[:, :N].T

    tab, stats = _tc_pre(x, degp2)
    dinv1d = tab[:, 5]
    dinv1d = jnp.asarray(dinv1d, jnp.float32).reshape(N)

    zeros8 = jnp.zeros((N, 8), jnp.float32)
    aggp, sp = _make_sc_l1()(tab, src3d, dst3d, dinv1d, zeros8)

    w1p = jnp.zeros((8, 64), jnp.float32).at[:5].set(W1)
    t0, t1, t2, t3 = _tc_mid(aggp, tab, w1p, b1.reshape(1, 64))

    agg2 = _make_sc_l2()(t0, t1, t2, t3, src3d, dst3d)

    ev = _tc_post(agg2[0], agg2[1], agg2[2], agg2[3], tab, sp[:, :N].T,
                  W2, b2.reshape(1, 64))

    tn = (T / T_max) * jnp.ones((1, 1), jnp.float32)
    out4, emb = _tc_head(ev, stats, tn, W3, b3.reshape(1, 16),
                         P1, pb1.reshape(1, 64), P2, pb2.reshape(1, 32),
                         P3, pb3.reshape(1, 4))
    return (out4, emb)


# speedup vs baseline: 53.3515x; 1.2272x over previous
import jax
import jax.numpy as jnp
from jax import lax
from jax.experimental import pallas as pl
from jax.experimental.pallas import tpu as pltpu
from jax.experimental.pallas import tpu_sc as plsc

N = 100000
E = 3200000
CH = 128
SCW = 2
NSCH = E // (SCW * CH)
BLK = 8
NBLK = NSCH // BLK
REM = NSCH - NBLK * BLK
NPAD = 102400
PT = NPAD // 16
NC, NS = 2, 16
NW = NC * NS
RCH = 800
NRC = N // RCH

_mesh = None


def _get_mesh():
    global _mesh
    if _mesh is None:
        _mesh = plsc.VectorSubcoreMesh(core_axis_name="c", subcore_axis_name="s")
    return _mesh


def _wait(src, dst, sem):
    pltpu.make_async_copy(src, dst, sem).wait()


def _sc_deg_body(dst3d, out, zbuf, idxb, ones, accum, s0, s1, s2, s3):
    sems = [s0, s1, s2, s3]
    c = lax.axis_index("c")
    s = lax.axis_index("s")
    wid = s * NC + c

    @pl.loop(0, PT // 16)
    def _z(i):
        zbuf[pl.ds(i * 16, 16)] = jnp.zeros((16,), jnp.float32)

    for m in range(SCW * CH // 16):
        ones[pl.ds(m * 16, 16)] = jnp.ones((16,), jnp.float32)

    ones2 = ones
    pltpu.sync_copy(zbuf, accum.at[pl.ds(s * PT, PT)])
    plsc.subcore_barrier()

    nblk_my = (NBLK - wid + NW - 1) // NW

    @pl.loop(0, nblk_my)
    def _blk(i):
        b0 = (wid + i * NW) * BLK
        pltpu.sync_copy(dst3d.at[pl.ds(b0, BLK)], idxb)
        for k in range(BLK):
            if k >= 4:
                _wait(ones2, accum.at[idxb.at[k - 4]], sems[k % 4])
            pltpu.async_copy(ones2, accum.at[idxb.at[k]], sems[k % 4], add=True)
        for k in range(BLK - 4, BLK):
            _wait(ones2, accum.at[idxb.at[k]], sems[k % 4])

    @pl.when(wid < REM)
    def _rem():
        pltpu.sync_copy(dst3d.at[NBLK * BLK + wid], idxb.at[0])
        pltpu.sync_copy(ones2, accum.at[idxb.at[0]], add=True)

    plsc.subcore_barrier()
    pltpu.sync_copy(accum.at[pl.ds(s * PT, PT)], out.at[c].at[pl.ds(s * PT, PT)])


def _sc_deg(dst3d):
    return pl.kernel(
        _sc_deg_body,
        out_type=jax.ShapeDtypeStruct((NC, NPAD), jnp.float32),
        mesh=_get_mesh(),
        compiler_params=pltpu.CompilerParams(use_tc_tiling_on_sc=False),
        scratch_types=[
            pltpu.VMEM((PT,), jnp.float32),
            pltpu.VMEM((BLK, SCW * CH), jnp.int32),
            pltpu.VMEM((SCW * CH,), jnp.float32),
            pltpu.VMEM_SHARED((NPAD,), jnp.float32),
        ] + [pltpu.SemaphoreType.DMA] * 4,
    )(dst3d)


def _make_sc_l1():
    def body(table0, src3d, dst3d, dinv1d, zeros8, aggp, sp,
             bufS, bufD, r0, r1, r2, r3, r4, r5, v0, v1, v2, v3, v4, v5, zbuf,
             dinv_sh, acc, sacc,
             g0, g1, g2, g3, g4, g5, c0s, c1s, c2s, c3s, c4s, c5s,
             e0, e1, e2, e3, e4, e5, f0, f1, f2, f3, f4, f5):
        rows = [r0, r1, r2, r3, r4, r5]
        valr = [v0, v1, v2, v3, v4, v5]
        gsem = [g0, g1, g2, g3, g4, g5]
        csem = [c0s, c1s, c2s, c3s, c4s, c5s]
        esem = [e0, e1, e2, e3, e4, e5]
        vsem = [f0, f1, f2, f3, f4, f5]
        c = lax.axis_index("c")
        s = lax.axis_index("s")
        wid = s * NC + c

        @pl.when(s == 0)
        def _stage():
            pltpu.sync_copy(dinv1d, dinv_sh)

        @pl.loop(0, PT // 16)
        def _z(i):
            zbuf[pl.ds(i * 16, 16)] = jnp.zeros((16,), jnp.float32)

        pltpu.sync_copy(zbuf, sacc.at[pl.ds(s * PT, PT)])

        @pl.loop(0, (NRC - s + NS - 1) // NS)
        def _init(i):
            r0_ = (s + i * NS) * RCH
            pltpu.sync_copy(zeros8.at[pl.ds(r0_, RCH)], acc.at[pl.ds(r0_, RCH)])

        plsc.subcore_barrier()

        def pipe(iS, iD):
            for k in range(BLK + 4):
                if k < BLK:
                    if k >= 6:
                        _wait(rows[k % 6], acc.at[iD.at[k - 6]], csem[k % 6])
                        _wait(valr[k % 6], sacc.at[iS.at[k - 6]], esem[k % 6])
                    pltpu.async_copy(dinv_sh.at[iD.at[k]], valr[k % 6],
                                     vsem[k % 6])
                    pltpu.async_copy(table0.at[iS.at[k]], rows[k % 6],
                                     gsem[k % 6])
                if k >= 4:
                    kk = k - 4
                    _wait(table0.at[iS.at[kk]], rows[kk % 6], gsem[kk % 6])
                    pltpu.async_copy(rows[kk % 6], acc.at[iD.at[kk]],
                                     csem[kk % 6], add=True)
                    _wait(dinv_sh.at[iD.at[kk]], valr[kk % 6], vsem[kk % 6])
                    pltpu.async_copy(valr[kk % 6], sacc.at[iS.at[kk]],
                                     esem[kk % 6], add=True)
            for k in range(BLK - 6, BLK):
                _wait(rows[k % 6], acc.at[iD.at[k]], csem[k % 6])
                _wait(valr[k % 6], sacc.at[iS.at[k]], esem[k % 6])

        nblk_my = (NBLK - wid + NW - 1) // NW

        @pl.loop(0, nblk_my)
        def _blk(i):
            b0 = (wid + i * NW) * BLK
            pltpu.sync_copy(src3d.at[pl.ds(b0, BLK)], bufS)
            pltpu.sync_copy(dst3d.at[pl.ds(b0, BLK)], bufD)
            pipe(bufS, bufD)

        @pl.when(wid < REM)
        def _rem():
            cix = NBLK * BLK + wid
            pltpu.sync_copy(src3d.at[cix], bufS.at[0])
            pltpu.sync_copy(dst3d.at[cix], bufD.at[0])
            pltpu.async_copy(dinv_sh.at[bufD.at[0]], valr[0], vsem[0]).wait()
            pltpu.sync_copy(valr[0], sacc.at[bufS.at[0]], add=True)
            pltpu.async_copy(table0.at[bufS.at[0]], rows[0], gsem[0]).wait()
            pltpu.sync_copy(rows[0], acc.at[bufD.at[0]], add=True)

        plsc.subcore_barrier()

        @pl.loop(0, (NRC - s + NS - 1) // NS)
        def _dump(i):
            r0_ = (s + i * NS) * RCH
            pltpu.sync_copy(acc.at[pl.ds(r0_, RCH)],
                            aggp.at[c].at[pl.ds(r0_, RCH)])

        pltpu.sync_copy(sacc.at[pl.ds(s * PT, PT)], sp.at[c].at[pl.ds(s * PT, PT)])

    return pl.kernel(
        body,
        out_type=(jax.ShapeDtypeStruct((NC, N, 8), jnp.float32),
                  jax.ShapeDtypeStruct((NC, NPAD), jnp.float32)),
        mesh=_get_mesh(),
        compiler_params=pltpu.CompilerParams(use_tc_tiling_on_sc=False),
        scratch_types=[
            pltpu.VMEM((BLK, SCW * CH), jnp.int32),
            pltpu.VMEM((BLK, SCW * CH), jnp.int32),
        ] + [pltpu.VMEM((SCW * CH, 8), jnp.float32)] * 6
          + [pltpu.VMEM((SCW * CH,), jnp.float32)] * 6
          + [
            pltpu.VMEM((PT,), jnp.float32),
            pltpu.VMEM_SHARED((N,), jnp.float32),
            pltpu.VMEM_SHARED((N, 8), jnp.float32),
            pltpu.VMEM_SHARED((NPAD,), jnp.float32),
        ] + [pltpu.SemaphoreType.DMA] * 24,
    )


def _make_sc_l2():
    def body(t0, t1, t2, t3, src3d, dst3d, out,
             bufS, bufD, r0, r1, r2, r3, r4, r5, acc,
             g0, g1, g2, g3, g4, g5, c0s, c1s, c2s, c3s, c4s, c5s):
        rows = [r0, r1, r2, r3, r4, r5]
        gsem = [g0, g1, g2, g3, g4, g5]
        csem = [c0s, c1s, c2s, c3s, c4s, c5s]
        c = lax.axis_index("c")
        s = lax.axis_index("s")
        tabs = [t0, t1, t2, t3]

        for g in range(4):
            tref = tabs[g]

            @pl.when(c == g // 2)
            def _grp(tref=tref, g=g):
                @pl.loop(0, (NRC - s + NS - 1) // NS)
                def _init(i):
                    r0_ = (s + i * NS) * RCH
                    pltpu.sync_copy(tref.at[pl.ds(r0_, RCH)],
                                    acc.at[pl.ds(r0_, RCH)])

                plsc.subcore_barrier()

                nblk_my = (NBLK - s + NS - 1) // NS

                @pl.loop(0, nblk_my)
                def _blk(i):
                    b0 = (s + i * NS) * BLK
                    pltpu.sync_copy(src3d.at[pl.ds(b0, BLK)], bufS)
                    pltpu.sync_copy(dst3d.at[pl.ds(b0, BLK)], bufD)
                    for k in range(BLK + 4):
                        if k < BLK:
                            if k >= 6:
                                _wait(rows[k % 6], acc.at[bufD.at[k - 6]],
                                      csem[k % 6])
                            pltpu.async_copy(tref.at[bufS.at[k]], rows[k % 6],
                                             gsem[k % 6])
                        if k >= 4:
                            kk = k - 4
                            _wait(tref.at[bufS.at[kk]], rows[kk % 6], gsem[kk % 6])
                            pltpu.async_copy(rows[kk % 6], acc.at[bufD.at[kk]],
                                             csem[kk % 6], add=True)
                    for k in range(BLK - 6, BLK):
                        _wait(rows[k % 6], acc.at[bufD.at[k]], csem[k % 6])

                @pl.when(s < REM)
                def _rem():
                    cix = NBLK * BLK + s
                    pltpu.sync_copy(src3d.at[cix], bufS.at[0])
                    pltpu.sync_copy(dst3d.at[cix], bufD.at[0])
                    pltpu.async_copy(tref.at[bufS.at[0]], rows[0], gsem[0]).wait()
                    pltpu.sync_copy(rows[0], acc.at[bufD.at[0]], add=True)

                plsc.subcore_barrier()

                @pl.loop(0, (NRC - s + NS - 1) // NS)
                def _dump(i):
                    r0_ = (s + i * NS) * RCH
                    pltpu.sync_copy(acc.at[pl.ds(r0_, RCH)],
                                    out.at[g].at[pl.ds(r0_, RCH)])

    return pl.kernel(
        body,
        out_type=jax.ShapeDtypeStruct((4, N, 16), jnp.float32),
        mesh=_get_mesh(),
        compiler_params=pltpu.CompilerParams(use_tc_tiling_on_sc=False,
                                             internal_scratch_in_bytes=1024 * 1024),
        scratch_types=[
            pltpu.VMEM((BLK, SCW * CH), jnp.int32),
            pltpu.VMEM((BLK, SCW * CH), jnp.int32),
        ] + [pltpu.VMEM((SCW * CH, 16), jnp.float32)] * 6
          + [pltpu.VMEM_SHARED((N, 16), jnp.float32)]
          + [pltpu.SemaphoreType.DMA] * 12,
    )


def _tc_pre_body(xT_ref, degp_ref, tabT_ref, stats_ref):
    xT = xT_ref[...]
    deg = degp_ref[0:1, :] + degp_ref[1:2, :] + 1.0
    dinv = lax.rsqrt(deg)
    tabT_ref[...] = jnp.concatenate(
        [xT * dinv, dinv, jnp.zeros((2, N), jnp.float32)], axis=0)
    m = (xT[2:3, :] == 1.0).astype(jnp.float32)
    vals = jnp.stack([
        jnp.sum(xT[2:3, :]), jnp.sum(xT[3:4, :]), jnp.sum(xT[4:5, :]),
        jnp.sum(xT[0:1, :] * m), jnp.sum(xT[1:2, :] * m), jnp.sum(m),
        jnp.float32(0.0), jnp.float32(0.0)])[None, :]
    stats_ref[...] = vals


def _tc_pre(xT, degp2):
    return pl.pallas_call(
        _tc_pre_body,
        out_shape=[
            jax.ShapeDtypeStruct((8, N), jnp.float32),
            jax.ShapeDtypeStruct((1, 8), jnp.float32),
        ],
    )(xT, degp2)


def _tc_mid_body(aggpT_ref, tabT_ref, w1t_ref, b1t_ref, y1T_ref):
    tabT = tabT_ref[...]
    aggT = aggpT_ref[0] + aggpT_ref[1] + tabT
    dinv = tabT[5:6, :]
    z = aggT * dinv
    h1 = jnp.maximum(jnp.dot(w1t_ref[...], z,
                             preferred_element_type=jnp.float32)
                     + b1t_ref[...], 0.0)
    y1T_ref[...] = h1 * dinv


def _tc_mid(aggpT, tabT, w1t, b1t):
    return pl.pallas_call(
        _tc_mid_body,
        out_shape=jax.ShapeDtypeStruct((64, N), jnp.float32),
    )(aggpT, tabT, w1t, b1t)


def _tc_post_body(agg2T_ref, tabT_ref, sp_ref, w2t_ref, b2t_ref, ev_ref):
    tabT = tabT_ref[...]
    dinv = tabT[5:6, :]
    z = agg2T_ref[...] * dinv
    h2 = jnp.maximum(jnp.dot(w2t_ref[...], z,
                             preferred_element_type=jnp.float32)
                     + b2t_ref[...], 0.0)
    sv = sp_ref[0:1, :] + sp_ref[1:2, :]
    cvec = dinv * (sv + dinv)
    ev_ref[...] = lax.dot_general(
        h2, cvec, (((1,), (1,)), ((), ())),
        preferred_element_type=jnp.float32)


def _tc_post(agg2T, tabT, sp2, w2t, b2t):
    return pl.pallas_call(
        _tc_post_body,
        out_shape=jax.ShapeDtypeStruct((64, 1), jnp.float32),
    )(agg2T, tabT, sp2, w2t, b2t)


def _tc_head_body(ev, stats, tn, w3, b3, p1, pb1, p2, pb2, p3, pb3,
                  out_ref, emb_ref):
    emb0 = jnp.dot(ev[...] * (1.0 / N), w3[...],
                   preferred_element_type=jnp.float32) + b3[...]
    st = stats[...]
    n_comp, n_and, n_or = st[0, 0], st[0, 1], st[0, 2]
    cnt = st[0, 5]
    avg_l = jnp.where(cnt > 0, st[0, 3] / jnp.maximum(cnt, 1.0), 0.0)
    avg_m = jnp.where(cnt > 0, st[0, 4] / jnp.maximum(cnt, 1.0), 0.0)
    tnv = tn[0, 0]
    gf = jnp.stack([n_comp, n_and, n_or, n_and + n_or, avg_l, avg_m,
                    tnv * 50.0, (1.0 / (1.0 + tnv)) * 50.0])[None, :]
    emb = jnp.concatenate([emb0, gf], axis=1)
    emb_ref[...] = emb
    h = jnp.maximum(jnp.dot(emb, p1[...], preferred_element_type=jnp.float32)
                    + pb1[...], 0.0)
    h = jnp.maximum(jnp.dot(h, p2[...], preferred_element_type=jnp.float32)
                    + pb2[...], 0.0)
    raw = jnp.dot(h, p3[...], preferred_element_type=jnp.float32) + pb3[...]
    z = raw + 2.0
    val = jnp.maximum(z, 0.0) + jnp.log1p(jnp.exp(-jnp.abs(z)))
    amin = 1.0 + val[:, 0:1]
    amax = amin + val[:, 1:2] + 0.5
    bmin = 1.0 + val[:, 2:3]
    bmax = bmin + val[:, 3:4] + 0.5
    out_ref[...] = jnp.concatenate([amin, amax, bmin, bmax], axis=1)


def _tc_head(ev, stats, tn, w3, b3_2d, p1, pb1_2d, p2, pb2_2d, p3, pb3_2d):
    return pl.pallas_call(
        _tc_head_body,
        out_shape=[
            jax.ShapeDtypeStruct((1, 4), jnp.float32),
            jax.ShapeDtypeStruct((1, 24), jnp.float32),
        ],
    )(ev, stats, tn, w3, b3_2d, p1, pb1_2d, p2, pb2_2d, p3, pb3_2d)


def kernel(x, edge_index, T, T_max, W1, b1, W2, b2, W3, b3,
           P1, pb1, P2, pb2, P3, pb3):
    src3d = edge_index[0].reshape(NSCH, SCW * CH)
    dst3d = edge_index[1].reshape(NSCH, SCW * CH)

    degp = _sc_deg(dst3d)
    degp2 = degp[:, :N]

    xT = x.T
    tabT, stats = _tc_pre(xT, degp2)
    dinv1d = tabT[5]
    tab_nm = tabT.T

    zeros8 = jnp.zeros((N, 8), jnp.float32)
    aggp, sp = _make_sc_l1()(tab_nm, src3d, dst3d, dinv1d, zeros8)
    aggpT = jnp.transpose(aggp, (0, 2, 1))

    w1t = jnp.zeros((64, 8), jnp.float32).at[:, :5].set(W1.T)
    y1T = _tc_mid(aggpT, tabT, w1t, b1.reshape(64, 1))

    t0 = y1T[0:16].T
    t1 = y1T[16:32].T
    t2 = y1T[32:48].T
    t3 = y1T[48:64].T
    agg2 = _make_sc_l2()(t0, t1, t2, t3, src3d, dst3d)
    agg2T = jnp.transpose(agg2, (0, 2, 1)).reshape(64, N)

    ev = _tc_post(agg2T, tabT, sp[:, :N], W2.T, b2.reshape(64, 1))

    tn = (T / T_max) * jnp.ones((1, 1), jnp.float32)
    out4, emb = _tc_head(ev.reshape(1, 64), stats, tn, W3, b3.reshape(1, 16),
                         P1, pb1.reshape(1, 64), P2, pb2.reshape(1, 32),
                         P3, pb3.reshape(1, 4))
    return (out4, emb)


# speedup vs baseline: 53.3679x; 1.0003x over previous
import jax
import jax.numpy as jnp
from jax import lax
from jax.experimental import pallas as pl
from jax.experimental.pallas import tpu as pltpu
from jax.experimental.pallas import tpu_sc as plsc

N = 100000
E = 3200000
CH = 128
SCW = 2
NSCH = E // (SCW * CH)
BLK = 8
NBLK = NSCH // BLK
REM = NSCH - NBLK * BLK
NPAD = 102400
PT = NPAD // 16
NC, NS = 2, 16
NW = NC * NS
RCH = 800
NRC = N // RCH

_mesh = None


def _get_mesh():
    global _mesh
    if _mesh is None:
        _mesh = plsc.VectorSubcoreMesh(core_axis_name="c", subcore_axis_name="s")
    return _mesh


def _wait(src, dst, sem):
    pltpu.make_async_copy(src, dst, sem).wait()


def _sc_deg_body(ei3, out, zbuf, idxb, ones, accum, s0, s1, s2, s3):
    dst3d = ei3.at[1]
    sems = [s0, s1, s2, s3]
    c = lax.axis_index("c")
    s = lax.axis_index("s")
    wid = s * NC + c

    @pl.loop(0, PT // 16)
    def _z(i):
        zbuf[pl.ds(i * 16, 16)] = jnp.zeros((16,), jnp.float32)

    for m in range(SCW * CH // 16):
        ones[pl.ds(m * 16, 16)] = jnp.ones((16,), jnp.float32)

    ones2 = ones
    pltpu.sync_copy(zbuf, accum.at[pl.ds(s * PT, PT)])
    plsc.subcore_barrier()

    nblk_my = (NBLK - wid + NW - 1) // NW

    @pl.loop(0, nblk_my)
    def _blk(i):
        b0 = (wid + i * NW) * BLK
        pltpu.sync_copy(dst3d.at[pl.ds(b0, BLK)], idxb)
        for k in range(BLK):
            if k >= 4:
                _wait(ones2, accum.at[idxb.at[k - 4]], sems[k % 4])
            pltpu.async_copy(ones2, accum.at[idxb.at[k]], sems[k % 4], add=True)
        for k in range(BLK - 4, BLK):
            _wait(ones2, accum.at[idxb.at[k]], sems[k % 4])

    @pl.when(wid < REM)
    def _rem():
        pltpu.sync_copy(dst3d.at[NBLK * BLK + wid], idxb.at[0])
        pltpu.sync_copy(ones2, accum.at[idxb.at[0]], add=True)

    plsc.subcore_barrier()
    pltpu.sync_copy(accum.at[pl.ds(s * PT, PT)], out.at[c].at[pl.ds(s * PT, PT)])


def _sc_deg(ei3):
    return pl.kernel(
        _sc_deg_body,
        out_type=jax.ShapeDtypeStruct((NC, NPAD), jnp.float32),
        mesh=_get_mesh(),
        compiler_params=pltpu.CompilerParams(use_tc_tiling_on_sc=False),
        scratch_types=[
            pltpu.VMEM((PT,), jnp.float32),
            pltpu.VMEM((BLK, SCW * CH), jnp.int32),
            pltpu.VMEM((SCW * CH,), jnp.float32),
            pltpu.VMEM_SHARED((NPAD,), jnp.float32),
        ] + [pltpu.SemaphoreType.DMA] * 4,
    )(ei3)


def _make_sc_l1():
    def body(table0, ei3, dinv1d, zeros8, aggp, sp,
             bufS, bufD, r0, r1, r2, r3, r4, r5, v0, v1, v2, v3, v4, v5, zbuf,
             dinv_sh, acc, sacc,
             g0, g1, g2, g3, g4, g5, c0s, c1s, c2s, c3s, c4s, c5s,
             e0, e1, e2, e3, e4, e5, f0, f1, f2, f3, f4, f5):
        src3d = ei3.at[0]
        dst3d = ei3.at[1]
        rows = [r0, r1, r2, r3, r4, r5]
        valr = [v0, v1, v2, v3, v4, v5]
        gsem = [g0, g1, g2, g3, g4, g5]
        csem = [c0s, c1s, c2s, c3s, c4s, c5s]
        esem = [e0, e1, e2, e3, e4, e5]
        vsem = [f0, f1, f2, f3, f4, f5]
        c = lax.axis_index("c")
        s = lax.axis_index("s")
        wid = s * NC + c

        @pl.when(s == 0)
        def _stage():
            pltpu.sync_copy(dinv1d, dinv_sh)

        @pl.loop(0, PT // 16)
        def _z(i):
            zbuf[pl.ds(i * 16, 16)] = jnp.zeros((16,), jnp.float32)

        pltpu.sync_copy(zbuf, sacc.at[pl.ds(s * PT, PT)])

        @pl.loop(0, (NRC - s + NS - 1) // NS)
        def _init(i):
            r0_ = (s + i * NS) * RCH
            pltpu.sync_copy(zeros8.at[pl.ds(r0_, RCH)], acc.at[pl.ds(r0_, RCH)])

        plsc.subcore_barrier()

        def pipe(iS, iD):
            for k in range(BLK + 4):
                if k < BLK:
                    if k >= 6:
                        _wait(rows[k % 6], acc.at[iD.at[k - 6]], csem[k % 6])
                        _wait(valr[k % 6], sacc.at[iS.at[k - 6]], esem[k % 6])
                    pltpu.async_copy(dinv_sh.at[iD.at[k]], valr[k % 6],
                                     vsem[k % 6])
                    pltpu.async_copy(table0.at[iS.at[k]], rows[k % 6],
                                     gsem[k % 6])
                if k >= 4:
                    kk = k - 4
                    _wait(table0.at[iS.at[kk]], rows[kk % 6], gsem[kk % 6])
                    pltpu.async_copy(rows[kk % 6], acc.at[iD.at[kk]],
                                     csem[kk % 6], add=True)
                    _wait(dinv_sh.at[iD.at[kk]], valr[kk % 6], vsem[kk % 6])
                    pltpu.async_copy(valr[kk % 6], sacc.at[iS.at[kk]],
                                     esem[kk % 6], add=True)
            for k in range(BLK - 6, BLK):
                _wait(rows[k % 6], acc.at[iD.at[k]], csem[k % 6])
                _wait(valr[k % 6], sacc.at[iS.at[k]], esem[k % 6])

        nblk_my = (NBLK - wid + NW - 1) // NW

        @pl.loop(0, nblk_my)
        def _blk(i):
            b0 = (wid + i * NW) * BLK
            pltpu.sync_copy(src3d.at[pl.ds(b0, BLK)], bufS)
            pltpu.sync_copy(dst3d.at[pl.ds(b0, BLK)], bufD)
            pipe(bufS, bufD)

        @pl.when(wid < REM)
        def _rem():
            cix = NBLK * BLK + wid
            pltpu.sync_copy(src3d.at[cix], bufS.at[0])
            pltpu.sync_copy(dst3d.at[cix], bufD.at[0])
            pltpu.async_copy(dinv_sh.at[bufD.at[0]], valr[0], vsem[0]).wait()
            pltpu.sync_copy(valr[0], sacc.at[bufS.at[0]], add=True)
            pltpu.async_copy(table0.at[bufS.at[0]], rows[0], gsem[0]).wait()
            pltpu.sync_copy(rows[0], acc.at[bufD.at[0]], add=True)

        plsc.subcore_barrier()

        @pl.loop(0, (NRC - s + NS - 1) // NS)
        def _dump(i):
            r0_ = (s + i * NS) * RCH
            pltpu.sync_copy(acc.at[pl.ds(r0_, RCH)],
                            aggp.at[c].at[pl.ds(r0_, RCH)])

        pltpu.sync_copy(sacc.at[pl.ds(s * PT, PT)], sp.at[c].at[pl.ds(s * PT, PT)])

    return pl.kernel(
        body,
        out_type=(jax.ShapeDtypeStruct((NC, N, 8), jnp.float32),
                  jax.ShapeDtypeStruct((NC, NPAD), jnp.float32)),
        mesh=_get_mesh(),
        compiler_params=pltpu.CompilerParams(use_tc_tiling_on_sc=False),
        scratch_types=[
            pltpu.VMEM((BLK, SCW * CH), jnp.int32),
            pltpu.VMEM((BLK, SCW * CH), jnp.int32),
        ] + [pltpu.VMEM((SCW * CH, 8), jnp.float32)] * 6
          + [pltpu.VMEM((SCW * CH,), jnp.float32)] * 6
          + [
            pltpu.VMEM((PT,), jnp.float32),
            pltpu.VMEM_SHARED((N,), jnp.float32),
            pltpu.VMEM_SHARED((N, 8), jnp.float32),
            pltpu.VMEM_SHARED((NPAD,), jnp.float32),
        ] + [pltpu.SemaphoreType.DMA] * 24,
    )


def _make_sc_l2():
    def body(t0, t1, t2, t3, ei3, out,
             bufS, bufD, r0, r1, r2, r3, r4, r5, acc,
             g0, g1, g2, g3, g4, g5, c0s, c1s, c2s, c3s, c4s, c5s):
        src3d = ei3.at[0]
        dst3d = ei3.at[1]
        rows = [r0, r1, r2, r3, r4, r5]
        gsem = [g0, g1, g2, g3, g4, g5]
        csem = [c0s, c1s, c2s, c3s, c4s, c5s]
        c = lax.axis_index("c")
        s = lax.axis_index("s")
        tabs = [t0, t1, t2, t3]

        for g in range(4):
            tref = tabs[g]

            @pl.when(c == g // 2)
            def _grp(tref=tref, g=g):
                @pl.loop(0, (NRC - s + NS - 1) // NS)
                def _init(i):
                    r0_ = (s + i * NS) * RCH
                    pltpu.sync_copy(tref.at[pl.ds(r0_, RCH)],
                                    acc.at[pl.ds(r0_, RCH)])

                plsc.subcore_barrier()

                nblk_my = (NBLK - s + NS - 1) // NS

                @pl.loop(0, nblk_my)
                def _blk(i):
                    b0 = (s + i * NS) * BLK
                    pltpu.sync_copy(src3d.at[pl.ds(b0, BLK)], bufS)
                    pltpu.sync_copy(dst3d.at[pl.ds(b0, BLK)], bufD)
                    for k in range(BLK + 4):
                        if k < BLK:
                            if k >= 6:
                                _wait(rows[k % 6], acc.at[bufD.at[k - 6]],
                                      csem[k % 6])
                            pltpu.async_copy(tref.at[bufS.at[k]], rows[k % 6],
                                             gsem[k % 6])
                        if k >= 4:
                            kk = k - 4
                            _wait(tref.at[bufS.at[kk]], rows[kk % 6], gsem[kk % 6])
                            pltpu.async_copy(rows[kk % 6], acc.at[bufD.at[kk]],
                                             csem[kk % 6], add=True)
                    for k in range(BLK - 6, BLK):
                        _wait(rows[k % 6], acc.at[bufD.at[k]], csem[k % 6])

                @pl.when(s < REM)
                def _rem():
                    cix = NBLK * BLK + s
                    pltpu.sync_copy(src3d.at[cix], bufS.at[0])
                    pltpu.sync_copy(dst3d.at[cix], bufD.at[0])
                    pltpu.async_copy(tref.at[bufS.at[0]], rows[0], gsem[0]).wait()
                    pltpu.sync_copy(rows[0], acc.at[bufD.at[0]], add=True)

                plsc.subcore_barrier()

                @pl.loop(0, (NRC - s + NS - 1) // NS)
                def _dump(i):
                    r0_ = (s + i * NS) * RCH
                    pltpu.sync_copy(acc.at[pl.ds(r0_, RCH)],
                                    out.at[g].at[pl.ds(r0_, RCH)])

    return pl.kernel(
        body,
        out_type=jax.ShapeDtypeStruct((4, N, 16), jnp.float32),
        mesh=_get_mesh(),
        compiler_params=pltpu.CompilerParams(use_tc_tiling_on_sc=False,
                                             internal_scratch_in_bytes=1024 * 1024),
        scratch_types=[
            pltpu.VMEM((BLK, SCW * CH), jnp.int32),
            pltpu.VMEM((BLK, SCW * CH), jnp.int32),
        ] + [pltpu.VMEM((SCW * CH, 16), jnp.float32)] * 6
          + [pltpu.VMEM_SHARED((N, 16), jnp.float32)]
          + [pltpu.SemaphoreType.DMA] * 12,
    )


def _tc_pre_body(xT_ref, degp_ref, tabT_ref, stats_ref):
    xT = xT_ref[...]
    deg = degp_ref[0:1, :] + degp_ref[1:2, :] + 1.0
    dinv = lax.rsqrt(deg)
    tabT_ref[...] = jnp.concatenate(
        [xT * dinv, dinv, jnp.zeros((2, N), jnp.float32)], axis=0)
    m = (xT[2:3, :] == 1.0).astype(jnp.float32)
    vals = jnp.stack([
        jnp.sum(xT[2:3, :]), jnp.sum(xT[3:4, :]), jnp.sum(xT[4:5, :]),
        jnp.sum(xT[0:1, :] * m), jnp.sum(xT[1:2, :] * m), jnp.sum(m),
        jnp.float32(0.0), jnp.float32(0.0)])[None, :]
    stats_ref[...] = vals


def _tc_pre(xT, degp2):
    return pl.pallas_call(
        _tc_pre_body,
        out_shape=[
            jax.ShapeDtypeStruct((8, N), jnp.float32),
            jax.ShapeDtypeStruct((1, 8), jnp.float32),
        ],
    )(xT, degp2)


def _tc_mid_body(aggpT_ref, tabT_ref, w1t_ref, b1t_ref, y1T_ref):
    tabT = tabT_ref[...]
    aggT = aggpT_ref[0] + aggpT_ref[1] + tabT
    dinv = tabT[5:6, :]
    z = aggT * dinv
    h1 = jnp.maximum(jnp.dot(w1t_ref[...], z,
                             preferred_element_type=jnp.float32)
                     + b1t_ref[...], 0.0)
    y1T_ref[...] = h1 * dinv


def _tc_mid(aggpT, tabT, w1t, b1t):
    return pl.pallas_call(
        _tc_mid_body,
        out_shape=jax.ShapeDtypeStruct((64, N), jnp.float32),
    )(aggpT, tabT, w1t, b1t)


def _tc_post_body(agg2T_ref, tabT_ref, sp_ref, w2t_ref, b2t_ref, ev_ref):
    tabT = tabT_ref[...]
    dinv = tabT[5:6, :]
    z = agg2T_ref[...] * dinv
    h2 = jnp.maximum(jnp.dot(w2t_ref[...], z,
                             preferred_element_type=jnp.float32)
                     + b2t_ref[...], 0.0)
    sv = sp_ref[0:1, :] + sp_ref[1:2, :]
    cvec = dinv * (sv + dinv)
    ev_ref[...] = lax.dot_general(
        h2, cvec, (((1,), (1,)), ((), ())),
        preferred_element_type=jnp.float32)


def _tc_post(agg2T, tabT, sp2, w2t, b2t):
    return pl.pallas_call(
        _tc_post_body,
        out_shape=jax.ShapeDtypeStruct((64, 1), jnp.float32),
    )(agg2T, tabT, sp2, w2t, b2t)


def _tc_head_body(ev, stats, tn, w3, b3, p1, pb1, p2, pb2, p3, pb3,
                  out_ref, emb_ref):
    emb0 = jnp.dot(ev[...] * (1.0 / N), w3[...],
                   preferred_element_type=jnp.float32) + b3[...]
    st = stats[...]
    n_comp, n_and, n_or = st[0, 0], st[0, 1], st[0, 2]
    cnt = st[0, 5]
    avg_l = jnp.where(cnt > 0, st[0, 3] / jnp.maximum(cnt, 1.0), 0.0)
    avg_m = jnp.where(cnt > 0, st[0, 4] / jnp.maximum(cnt, 1.0), 0.0)
    tnv = tn[0, 0]
    gf = jnp.stack([n_comp, n_and, n_or, n_and + n_or, avg_l, avg_m,
                    tnv * 50.0, (1.0 / (1.0 + tnv)) * 50.0])[None, :]
    emb = jnp.concatenate([emb0, gf], axis=1)
    emb_ref[...] = emb
    h = jnp.maximum(jnp.dot(emb, p1[...], preferred_element_type=jnp.float32)
                    + pb1[...], 0.0)
    h = jnp.maximum(jnp.dot(h, p2[...], preferred_element_type=jnp.float32)
                    + pb2[...], 0.0)
    raw = jnp.dot(h, p3[...], preferred_element_type=jnp.float32) + pb3[...]
    z = raw + 2.0
    val = jnp.maximum(z, 0.0) + jnp.log1p(jnp.exp(-jnp.abs(z)))
    amin = 1.0 + val[:, 0:1]
    amax = amin + val[:, 1:2] + 0.5
    bmin = 1.0 + val[:, 2:3]
    bmax = bmin + val[:, 3:4] + 0.5
    out_ref[...] = jnp.concatenate([amin, amax, bmin, bmax], axis=1)


def _tc_head(ev, stats, tn, w3, b3_2d, p1, pb1_2d, p2, pb2_2d, p3, pb3_2d):
    return pl.pallas_call(
        _tc_head_body,
        out_shape=[
            jax.ShapeDtypeStruct((1, 4), jnp.float32),
            jax.ShapeDtypeStruct((1, 24), jnp.float32),
        ],
    )(ev, stats, tn, w3, b3_2d, p1, pb1_2d, p2, pb2_2d, p3, pb3_2d)


def kernel(x, edge_index, T, T_max, W1, b1, W2, b2, W3, b3,
           P1, pb1, P2, pb2, P3, pb3):
    ei3 = edge_index.reshape(2, NSCH, SCW * CH)

    degp = _sc_deg(ei3)
    degp2 = degp[:, :N]

    xT = x.T
    tabT, stats = _tc_pre(xT, degp2)
    dinv1d = tabT[5]
    tab_nm = tabT.T

    zeros8 = jnp.zeros((N, 8), jnp.float32)
    aggp, sp = _make_sc_l1()(tab_nm, ei3, dinv1d, zeros8)
    aggpT = jnp.transpose(aggp, (0, 2, 1))

    w1t = jnp.zeros((64, 8), jnp.float32).at[:, :5].set(W1.T)
    y1T = _tc_mid(aggpT, tabT, w1t, b1.reshape(64, 1))

    t0 = y1T[0:16].T
    t1 = y1T[16:32].T
    t2 = y1T[32:48].T
    t3 = y1T[48:64].T
    agg2 = _make_sc_l2()(t0, t1, t2, t3, ei3)
    agg2T = jnp.transpose(agg2, (0, 2, 1)).reshape(64, N)

    ev = _tc_post(agg2T, tabT, sp[:, :N], W2.T, b2.reshape(64, 1))

    tn = (T / T_max) * jnp.ones((1, 1), jnp.float32)
    out4, emb = _tc_head(ev.reshape(1, 64), stats, tn, W3, b3.reshape(1, 16),
                         P1, pb1.reshape(1, 64), P2, pb2.reshape(1, 32),
                         P3, pb3.reshape(1, 4))
    return (out4, emb)
